# Initial kernel scaffold; baseline (speedup 1.0000x reference)
import jax
import jax.numpy as jnp
from jax.experimental import pallas as pl


def kernel(predictions, targets):
    raise NotImplementedError("write your pallas kernel here")


# speedup vs baseline: 75.7091x; 75.7091x over previous
import functools
import math

import jax
import jax.numpy as jnp
from jax import lax
from jax.experimental import pallas as pl
from jax.experimental.pallas import tpu as pltpu
from jax.experimental.pallas import tpu_sc as plsc

B = 256
L = 200
LP = 208
NBLK = LP // 16
K = 10
SIGMA = 1.0
NC = 2
NS = 16
NW = NC * NS
EX_PER_W = B // NW
LN2 = 0.6931471805599453
PRED_PAD = -3.0e38
TARG_PAD = -100.0

_DISC_LIST = [1.0 / math.log2(r + 2.0) for r in range(K)] + [0.0] * (16 - K)


def _top16(ref, lane):
    rk = ref[pl.ds(0, 16)]
    rv = lane
    rk, rv = plsc.sort_key_val(rk, rv, descending=True)
    for b in range(1, NBLK):
        kb = ref[pl.ds(16 * b, 16)]
        vb = lane + 16 * b
        kb, vb = plsc.sort_key_val(kb, vb, descending=True)
        kbr = lax.rev(kb, (0,))
        vbr = lax.rev(vb, (0,))
        take = rk >= kbr
        mk = jnp.where(take, rk, kbr)
        mv = jnp.where(take, rv, vbr)
        rk, rv = plsc.sort_key_val(mk, mv, descending=True)
    return rk, rv


def _sc_body(pred_hbm, targ_hbm, out_hbm, pred_v, targ_v, g_v, di_v, out_v):
    lane = lax.iota(jnp.int32, 16)
    zeros16 = (lane ^ lane).astype(jnp.float32)
    disc = zeros16
    for r in range(K):
        disc = jnp.where(lane == r, _DISC_LIST[r], disc)
    wid = lax.axis_index("s") * NC + lax.axis_index("c")

    def ex_body(e, total):
        row = wid * EX_PER_W + e
        pltpu.sync_copy(pred_hbm.at[row], pred_v)
        pltpu.sync_copy(targ_hbm.at[row], targ_v)
        for b in range(NBLK):
            sl = pl.ds(16 * b, 16)
            g_v[sl] = jnp.exp(targ_v[sl] * LN2) - 1.0
            di_v[sl] = zeros16
        pk, pv = _top16(pred_v, lane)
        plsc.store_scatter(di_v, [pv], disc, mask=lane < K)
        gk, _ = _top16(g_v, lane)
        ideal = jnp.broadcast_to(jnp.sum(gk * disc), (16,))
        recip = jnp.where(ideal == 0.0, zeros16, 1.0 / ideal)
        g_top = plsc.load_gather(g_v, [pv])

        def t_body(t, accs):
            acc_m, acc_c = accs
            sel = lane == t
            gt = jnp.sum(jnp.where(sel, g_top, 0.0))
            pt = jnp.sum(jnp.where(sel, pk, 0.0))
            dt = jnp.sum(jnp.where(sel, disc, 0.0))
            for b in range(NBLK):
                sl = pl.ds(16 * b, 16)
                gj = g_v[sl]
                pj = pred_v[sl]
                dj = di_v[sl]
                gd = gt - gj
                pd = pt - pj
                mis = ((gd > 0) & (pd < 0)) | ((gd < 0) & (pd > 0))
                val = jnp.abs(gd * (dj - dt)) * (pt + pj)
                acc_m = acc_m + jnp.where(mis, val, 0.0)
            gd = gt - g_top
            pd = pt - pk
            mis = ((gd > 0) & (pd < 0)) | ((gd < 0) & (pd > 0))
            val = jnp.abs(gd * (disc - dt)) * (pt + pk)
            acc_c = acc_c + jnp.where(mis & (lane < K), val, 0.0)
            return acc_m, acc_c

        acc_m, acc_c = lax.fori_loop(0, K, t_body, (zeros16, zeros16))
        return total + (acc_m - 0.5 * acc_c) * (-SIGMA * recip)

    total = lax.fori_loop(0, EX_PER_W, ex_body, zeros16)
    out_v[...] = total
    pltpu.sync_copy(out_v, out_hbm.at[wid])


_sc_kernel = functools.partial(
    pl.kernel,
    out_type=jax.ShapeDtypeStruct((NW, 16), jnp.float32),
    mesh=plsc.VectorSubcoreMesh(
        core_axis_name="c", subcore_axis_name="s", num_cores=NC, num_subcores=NS
    ),
    compiler_params=pltpu.CompilerParams(needs_layout_passes=False),
    scratch_types=[
        pltpu.VMEM((LP,), jnp.float32),
        pltpu.VMEM((LP,), jnp.float32),
        pltpu.VMEM((LP,), jnp.float32),
        pltpu.VMEM((LP,), jnp.float32),
        pltpu.VMEM((16,), jnp.float32),
    ],
)(_sc_body)


def kernel(predictions, targets):
    pred_pad = jnp.full((B, LP), PRED_PAD, jnp.float32).at[:, :L].set(predictions)
    targ_pad = jnp.full((B, LP), TARG_PAD, jnp.float32).at[:, :L].set(targets)
    partials = _sc_kernel(pred_pad, targ_pad)
    return jnp.sum(partials) / B


# speedup vs baseline: 119.3184x; 1.5760x over previous
import functools
import math

import jax
import jax.numpy as jnp
from jax import lax
from jax.experimental import pallas as pl
from jax.experimental.pallas import tpu as pltpu
from jax.experimental.pallas import tpu_sc as plsc

B = 256
L = 200
LP = 208
NBLK = LP // 16
K = 10
SIGMA = 1.0
NC = 2
NS = 16
NW = NC * NS
EX_PER_W = B // NW
LN2 = 0.6931471805599453
PRED_PAD = -3.0e38
TARG_PAD = -100.0

_DISC_LIST = [1.0 / math.log2(r + 2.0) for r in range(K)] + [0.0] * (16 - K)
_IN_BOUNDS = "promise_in_bounds"


def _merge16(a, b):
    ak, av = a
    bk, bv = b
    bkr = lax.rev(bk, (0,))
    bvr = lax.rev(bv, (0,))
    take = ak >= bkr
    mk = jnp.where(take, ak, bkr)
    mv = jnp.where(take, av, bvr)
    return plsc.sort_key_val(mk, mv, descending=True)


def _top16(kv_blocks):
    runs = [plsc.sort_key_val(k, v, descending=True) for k, v in kv_blocks]
    while len(runs) > 1:
        nxt = [_merge16(runs[i], runs[i + 1]) for i in range(0, len(runs) - 1, 2)]
        if len(runs) % 2:
            nxt.append(runs[-1])
        runs = nxt
    return runs[0]


def _sc_body(pred_hbm, targ_hbm, out_hbm,
             pred_v, targ_v, g_v, di_v, out_v):
    lane = lax.iota(jnp.int32, 16)
    izero = lane ^ lane
    zeros16 = izero.astype(jnp.float32)
    disc = zeros16
    for r in range(K):
        disc = jnp.where(lane == r, _DISC_LIST[r], disc)
    wid = lax.axis_index("s") * NC + lax.axis_index("c")

    def ex_body(e, total):
        row = wid * EX_PER_W + e
        pltpu.sync_copy(pred_hbm.at[row], pred_v)
        pltpu.sync_copy(targ_hbm.at[row], targ_v)
        pj = [pred_v[pl.ds(16 * b, 16)] for b in range(NBLK)]
        gj = [jnp.exp(targ_v[pl.ds(16 * b, 16)] * LN2) - 1.0
              for b in range(NBLK)]
        for b in range(NBLK):
            sl = pl.ds(16 * b, 16)
            g_v[sl] = gj[b]
            di_v[sl] = zeros16
        pk, pv = _top16([(pj[b], lane + 16 * b) for b in range(NBLK)])
        plsc.store_scatter(di_v, [pv], disc, mask=lane < K)
        gk, _ = _top16([(gj[b], lane + 16 * b) for b in range(NBLK)])
        ideal = jnp.broadcast_to(jnp.sum(gk * disc), (16,))
        recip = jnp.where(ideal == 0.0, zeros16, 1.0 / ideal)
        g_top = plsc.load_gather(g_v, [pv])
        dj = [di_v[pl.ds(16 * b, 16)] for b in range(NBLK)]

        acc_m = zeros16
        acc_c = zeros16
        for t in range(K):
            sel = lane == t
            gt = jnp.broadcast_to(jnp.sum(jnp.where(sel, g_top, 0.0)), (16,))
            pt = jnp.broadcast_to(jnp.sum(jnp.where(sel, pk, 0.0)), (16,))
            dt = _DISC_LIST[t]
            for b in range(NBLK):
                gd = gt - gj[b]
                pd = pt - pj[b]
                mis = gd * pd < 0.0
                val = jnp.abs(gd * (dj[b] - dt)) * (pt + pj[b])
                acc_m = acc_m + jnp.where(mis, val, 0.0)
            gd = gt - g_top
            pd = pt - pk
            mis = (gd * pd < 0.0) & (lane < K)
            val = jnp.abs(gd * (disc - dt)) * (pt + pk)
            acc_c = acc_c + jnp.where(mis, val, 0.0)

        return total + (acc_m - 0.5 * acc_c) * (-SIGMA * recip)

    total = lax.fori_loop(0, EX_PER_W, ex_body, zeros16)
    out_v[...] = total
    pltpu.sync_copy(out_v, out_hbm.at[wid])


_sc_kernel = functools.partial(
    pl.kernel,
    out_type=jax.ShapeDtypeStruct((NW, 16), jnp.float32),
    mesh=plsc.VectorSubcoreMesh(
        core_axis_name="c", subcore_axis_name="s", num_cores=NC, num_subcores=NS
    ),
    compiler_params=pltpu.CompilerParams(needs_layout_passes=False),
    scratch_types=[
        pltpu.VMEM((LP,), jnp.float32),
        pltpu.VMEM((LP,), jnp.float32),
        pltpu.VMEM((LP,), jnp.float32),
        pltpu.VMEM((LP,), jnp.float32),
        pltpu.VMEM((16,), jnp.float32),
    ],
)(_sc_body)


def kernel(predictions, targets):
    pred_pad = jnp.full((B, LP), PRED_PAD, jnp.float32).at[:, :L].set(predictions)
    targ_pad = jnp.full((B, LP), TARG_PAD, jnp.float32).at[:, :L].set(targets)
    partials = _sc_kernel(pred_pad, targ_pad)
    return jnp.sum(partials) / B


# speedup vs baseline: 147.4604x; 1.2359x over previous
import functools
import math

import jax
import jax.numpy as jnp
from jax import lax
from jax.experimental import pallas as pl
from jax.experimental.pallas import tpu as pltpu
from jax.experimental.pallas import tpu_sc as plsc

B = 256
L = 200
LP = 208
NBLK = LP // 16
K = 10
SIGMA = 1.0
NC = 2
NS = 16
NW = NC * NS
EX_PER_W = B // NW
LN2 = 0.6931471805599453
PRED_PAD = -3.0e38
TARG_PAD = -100.0

_DISC_LIST = [1.0 / math.log2(r + 2.0) for r in range(K)] + [0.0] * (16 - K)
_IN_BOUNDS = "promise_in_bounds"


def _merge16(a, b):
    ak, av = a
    bk, bv = b
    bkr = lax.rev(bk, (0,))
    bvr = lax.rev(bv, (0,))
    take = ak >= bkr
    mk = jnp.where(take, ak, bkr)
    mv = jnp.where(take, av, bvr)
    return plsc.sort_key_val(mk, mv, descending=True)


def _top16(kv_blocks):
    runs = [plsc.sort_key_val(k, v, descending=True) for k, v in kv_blocks]
    while len(runs) > 1:
        nxt = [_merge16(runs[i], runs[i + 1]) for i in range(0, len(runs) - 1, 2)]
        if len(runs) % 2:
            nxt.append(runs[-1])
        runs = nxt
    return runs[0]


def _sc_body(pred_hbm, targ_hbm, out_hbm,
             pred8_v, targ8_v, g_v, di_v, out_v):
    lane = lax.iota(jnp.int32, 16)
    izero = lane ^ lane
    zeros16 = izero.astype(jnp.float32)
    disc = zeros16
    for r in range(K):
        disc = jnp.where(lane == r, _DISC_LIST[r], disc)
    wid = lax.axis_index("s") * NC + lax.axis_index("c")

    pltpu.sync_copy(pred_hbm.at[pl.ds(wid * EX_PER_W, EX_PER_W)], pred8_v)
    pltpu.sync_copy(targ_hbm.at[pl.ds(wid * EX_PER_W, EX_PER_W)], targ8_v)

    def ex_body(e, total):
        pj = [pred8_v[e, pl.ds(16 * b, 16)] for b in range(NBLK)]
        gj = [jnp.exp(targ8_v[e, pl.ds(16 * b, 16)] * LN2) - 1.0
              for b in range(NBLK)]
        for b in range(NBLK):
            sl = pl.ds(16 * b, 16)
            g_v[sl] = gj[b]
            di_v[sl] = zeros16
        pk, pv = _top16([(pj[b], lane + 16 * b) for b in range(NBLK)])
        plsc.store_scatter(di_v, [pv], disc, mask=lane < K)
        gk, _ = _top16([(gj[b], lane + 16 * b) for b in range(NBLK)])
        ideal = jnp.broadcast_to(jnp.sum(gk * disc), (16,))
        recip = jnp.where(ideal == 0.0, zeros16, 1.0 / ideal)
        g_top = plsc.load_gather(g_v, [pv])
        dj = [di_v[pl.ds(16 * b, 16)] for b in range(NBLK)]

        acc_m = zeros16
        acc_c = zeros16
        for t in range(K):
            sel = lane == t
            gt = jnp.broadcast_to(jnp.sum(jnp.where(sel, g_top, 0.0)), (16,))
            pt = jnp.broadcast_to(jnp.sum(jnp.where(sel, pk, 0.0)), (16,))
            dt = _DISC_LIST[t]
            acc_t = zeros16
            for b in range(NBLK):
                gd = gt - gj[b]
                pd = pt - pj[b]
                mis = gd * pd < 0.0
                val = jnp.abs(gd * (dj[b] - dt)) * (pt + pj[b])
                acc_t = acc_t + jnp.where(mis, val, 0.0)
            acc_m = acc_m + acc_t
            gd = gt - g_top
            pd = pt - pk
            mis = (gd * pd < 0.0) & (lane < K)
            val = jnp.abs(gd * (disc - dt)) * (pt + pk)
            acc_c = acc_c + jnp.where(mis, val, 0.0)

        return total + (acc_m - 0.5 * acc_c) * (-SIGMA * recip)

    total = lax.fori_loop(0, EX_PER_W, ex_body, zeros16)
    out_v[...] = total
    pltpu.sync_copy(out_v, out_hbm.at[wid])


_sc_kernel = functools.partial(
    pl.kernel,
    out_type=jax.ShapeDtypeStruct((NW, 16), jnp.float32),
    mesh=plsc.VectorSubcoreMesh(
        core_axis_name="c", subcore_axis_name="s", num_cores=NC, num_subcores=NS
    ),
    compiler_params=pltpu.CompilerParams(needs_layout_passes=False),
    scratch_types=[
        pltpu.VMEM((EX_PER_W, LP), jnp.float32),
        pltpu.VMEM((EX_PER_W, LP), jnp.float32),
        pltpu.VMEM((LP,), jnp.float32),
        pltpu.VMEM((LP,), jnp.float32),
        pltpu.VMEM((16,), jnp.float32),
    ],
)(_sc_body)


def kernel(predictions, targets):
    pred_pad = jnp.full((B, LP), PRED_PAD, jnp.float32).at[:, :L].set(predictions)
    targ_pad = jnp.full((B, LP), TARG_PAD, jnp.float32).at[:, :L].set(targets)
    partials = _sc_kernel(pred_pad, targ_pad)
    return jnp.sum(partials) / B


# speedup vs baseline: 148.9371x; 1.0100x over previous
import functools
import math

import jax
import jax.numpy as jnp
from jax import lax
from jax.experimental import pallas as pl
from jax.experimental.pallas import tpu as pltpu
from jax.experimental.pallas import tpu_sc as plsc

B = 256
L = 200
NBLK = 13
TAIL_OFF = L - 16
K = 10
SIGMA = 1.0
NC = 2
NS = 16
NW = NC * NS
EX_PER_W = B // NW
LN2 = 0.6931471805599453
NEG_HUGE = -3.0e38

_DISC_LIST = [1.0 / math.log2(r + 2.0) for r in range(K)] + [0.0] * (16 - K)


def _blk(b):
    return pl.ds(16 * b if b < NBLK - 1 else TAIL_OFF, 16)


def _merge16(a, b):
    ak, av = a
    bk, bv = b
    bkr = lax.rev(bk, (0,))
    bvr = lax.rev(bv, (0,))
    take = ak >= bkr
    mk = jnp.where(take, ak, bkr)
    mv = jnp.where(take, av, bvr)
    return plsc.sort_key_val(mk, mv, descending=True)


def _top16(kv_blocks):
    runs = [plsc.sort_key_val(k, v, descending=True) for k, v in kv_blocks]
    while len(runs) > 1:
        nxt = [_merge16(runs[i], runs[i + 1]) for i in range(0, len(runs) - 1, 2)]
        if len(runs) % 2:
            nxt.append(runs[-1])
        runs = nxt
    return runs[0]


def _sc_body(pred_hbm, targ_hbm, out_hbm, pred8_v, targ8_v, g_v, di_v, out_v):
    lane = lax.iota(jnp.int32, 16)
    zeros16 = (lane ^ lane).astype(jnp.float32)
    disc = zeros16
    for r in range(K):
        disc = jnp.where(lane == r, _DISC_LIST[r], disc)
    tail_ok = lane >= (16 * (NBLK - 1) - TAIL_OFF)
    wid = lax.axis_index("s") * NC + lax.axis_index("c")

    pltpu.sync_copy(pred_hbm.at[pl.ds(wid * EX_PER_W, EX_PER_W)], pred8_v)
    pltpu.sync_copy(targ_hbm.at[pl.ds(wid * EX_PER_W, EX_PER_W)], targ8_v)

    def ex_body(e, total):
        pj = [pred8_v[e, _blk(b)] for b in range(NBLK)]
        gj = [jnp.exp(targ8_v[e, _blk(b)] * LN2) - 1.0 for b in range(NBLK)]
        for b in range(NBLK):
            g_v[_blk(b)] = gj[b]
            di_v[_blk(b)] = zeros16
        pkey = pj[:-1] + [jnp.where(tail_ok, pj[-1], NEG_HUGE)]
        gkey = gj[:-1] + [jnp.where(tail_ok, gj[-1], NEG_HUGE)]
        item = [lane + 16 * b for b in range(NBLK - 1)] + [lane + TAIL_OFF]
        pk, pv = _top16([(pkey[b], item[b]) for b in range(NBLK)])
        plsc.store_scatter(di_v, [pv], disc, mask=lane < K)
        gk, _ = _top16([(gkey[b], item[b]) for b in range(NBLK)])
        ideal = jnp.broadcast_to(jnp.sum(gk * disc), (16,))
        recip = jnp.where(ideal == 0.0, zeros16, 1.0 / ideal)
        g_top = plsc.load_gather(g_v, [pv])
        dj = [di_v[_blk(b)] for b in range(NBLK)]

        acc_m = zeros16
        acc_c = zeros16
        for t in range(K):
            sel = lane == t
            gt = jnp.broadcast_to(jnp.sum(jnp.where(sel, g_top, 0.0)), (16,))
            pt = jnp.broadcast_to(jnp.sum(jnp.where(sel, pk, 0.0)), (16,))
            dt = _DISC_LIST[t]
            acc_t = zeros16
            for b in range(NBLK):
                gd = gt - gj[b]
                pd = pt - pj[b]
                mis = gd * pd < 0.0
                if b == NBLK - 1:
                    mis = mis & tail_ok
                val = jnp.abs(gd * (dj[b] - dt)) * (pt + pj[b])
                acc_t = acc_t + jnp.where(mis, val, 0.0)
            acc_m = acc_m + acc_t
            gd = gt - g_top
            pd = pt - pk
            mis = (gd * pd < 0.0) & (lane < K)
            val = jnp.abs(gd * (disc - dt)) * (pt + pk)
            acc_c = acc_c + jnp.where(mis, val, 0.0)

        return total + (acc_m - 0.5 * acc_c) * (-SIGMA * recip)

    total = lax.fori_loop(0, EX_PER_W, ex_body, zeros16)
    out_v[...] = total
    pltpu.sync_copy(out_v, out_hbm.at[wid])


_sc_kernel = functools.partial(
    pl.kernel,
    out_type=jax.ShapeDtypeStruct((NW, 16), jnp.float32),
    mesh=plsc.VectorSubcoreMesh(
        core_axis_name="c", subcore_axis_name="s", num_cores=NC, num_subcores=NS
    ),
    compiler_params=pltpu.CompilerParams(needs_layout_passes=False),
    scratch_types=[
        pltpu.VMEM((EX_PER_W, L), jnp.float32),
        pltpu.VMEM((EX_PER_W, L), jnp.float32),
        pltpu.VMEM((L,), jnp.float32),
        pltpu.VMEM((L,), jnp.float32),
        pltpu.VMEM((16,), jnp.float32),
    ],
)(_sc_body)


def kernel(predictions, targets):
    partials = _sc_kernel(predictions, targets)
    return jnp.sum(partials) / B


# speedup vs baseline: 152.2455x; 1.0222x over previous
import functools
import math

import jax
import jax.numpy as jnp
from jax import lax
from jax.experimental import pallas as pl
from jax.experimental.pallas import tpu as pltpu
from jax.experimental.pallas import tpu_sc as plsc

B = 256
L = 200
NBLK = 13
TAIL_OFF = L - 16
K = 10
SIGMA = 1.0
NC = 2
NS = 16
NW = NC * NS
EX_PER_W = B // NW
LN2 = 0.6931471805599453
NEG_HUGE = -3.0e38

_DISC_LIST = [1.0 / math.log2(r + 2.0) for r in range(K)] + [0.0] * (16 - K)


def _blk(b):
    return pl.ds(16 * b if b < NBLK - 1 else TAIL_OFF, 16)


def _merge16(a, b):
    ak, av = a
    bk, bv = b
    bkr = lax.rev(bk, (0,))
    bvr = lax.rev(bv, (0,))
    take = ak >= bkr
    mk = jnp.where(take, ak, bkr)
    mv = jnp.where(take, av, bvr)
    return plsc.sort_key_val(mk, mv, descending=True)


def _top16(kv_blocks):
    runs = [plsc.sort_key_val(k, v, descending=True) for k, v in kv_blocks]
    while len(runs) > 1:
        nxt = [_merge16(runs[i], runs[i + 1]) for i in range(0, len(runs) - 1, 2)]
        if len(runs) % 2:
            nxt.append(runs[-1])
        runs = nxt
    return runs[0]


def _top16_vals_asc(key_blocks):
    runs = [jnp.sort(kb) for kb in key_blocks]
    while len(runs) > 1:
        nxt = []
        for i in range(0, len(runs) - 1, 2):
            mk = jnp.minimum(runs[i], lax.rev(runs[i + 1], (0,)))
            nxt.append(jnp.sort(mk))
        if len(runs) % 2:
            nxt.append(runs[-1])
        runs = nxt
    return runs[0]


def _sc_body(pred_hbm, targ_hbm, out_hbm, pred8_v, targ8_v, g_v, out_v):
    lane = lax.iota(jnp.int32, 16)
    zeros16 = (lane ^ lane).astype(jnp.float32)
    disc = zeros16
    for r in range(K):
        disc = jnp.where(lane == r, _DISC_LIST[r], disc)
    tail_ok = lane >= (16 * (NBLK - 1) - TAIL_OFF)
    wid = lax.axis_index("s") * NC + lax.axis_index("c")

    pltpu.sync_copy(pred_hbm.at[pl.ds(wid * EX_PER_W, EX_PER_W)], pred8_v)
    pltpu.sync_copy(targ_hbm.at[pl.ds(wid * EX_PER_W, EX_PER_W)], targ8_v)

    def ex_body(e, total):
        pj = [pred8_v[e, _blk(b)] for b in range(NBLK)]
        gj = [jnp.exp(targ8_v[e, _blk(b)] * LN2) - 1.0 for b in range(NBLK)]
        for b in range(NBLK):
            g_v[_blk(b)] = gj[b]
        pkey = pj[:-1] + [jnp.where(tail_ok, pj[-1], NEG_HUGE)]
        item = [lane + 16 * b for b in range(NBLK - 1)] + [lane + TAIL_OFF]
        pk, pv = _top16([(pkey[b], item[b]) for b in range(NBLK)])
        gkey = [-gj[b] for b in range(NBLK - 1)] + [
            jnp.where(tail_ok, -gj[-1], -NEG_HUGE)]
        gk_neg = _top16_vals_asc(gkey)
        ideal = jnp.broadcast_to(-jnp.sum(gk_neg * disc), (16,))
        recip = jnp.where(ideal == 0.0, zeros16, 1.0 / ideal)
        g_top = plsc.load_gather(g_v, [pv])

        acc = zeros16
        for t in range(K):
            sel = lane == t
            gt = jnp.broadcast_to(jnp.sum(jnp.where(sel, g_top, 0.0)), (16,))
            pt = jnp.broadcast_to(jnp.sum(jnp.where(sel, pk, 0.0)), (16,))
            dt = _DISC_LIST[t]
            sub_t = zeros16
            for b in range(NBLK):
                gd = gt - gj[b]
                pd = pt - pj[b]
                mis = gd * pd < 0.0
                if b == NBLK - 1:
                    mis = mis & tail_ok
                val = jnp.abs(gd) * (pt + pj[b])
                sub_t = sub_t + jnp.where(mis, val, 0.0)
            acc = acc + dt * sub_t
            gd = gt - g_top
            pd = pt - pk
            mis = (gd * pd < 0.0) & (lane < K)
            cval = (0.5 * jnp.abs(gd * (disc - dt)) - jnp.abs(gd) * dt) * (pt + pk)
            acc = acc + jnp.where(mis, cval, 0.0)

        return total + acc * (-SIGMA * recip)

    total = lax.fori_loop(0, EX_PER_W, ex_body, zeros16)
    out_v[...] = total
    pltpu.sync_copy(out_v, out_hbm.at[wid])


_sc_kernel = functools.partial(
    pl.kernel,
    out_type=jax.ShapeDtypeStruct((NW, 16), jnp.float32),
    mesh=plsc.VectorSubcoreMesh(
        core_axis_name="c", subcore_axis_name="s", num_cores=NC, num_subcores=NS
    ),
    compiler_params=pltpu.CompilerParams(needs_layout_passes=False),
    scratch_types=[
        pltpu.VMEM((EX_PER_W, L), jnp.float32),
        pltpu.VMEM((EX_PER_W, L), jnp.float32),
        pltpu.VMEM((L,), jnp.float32),
        pltpu.VMEM((16,), jnp.float32),
    ],
)(_sc_body)


def kernel(predictions, targets):
    partials = _sc_kernel(predictions, targets)
    return jnp.sum(partials) / B


# speedup vs baseline: 152.4681x; 1.0015x over previous
import functools
import math

import jax
import jax.numpy as jnp
from jax import lax
from jax.experimental import pallas as pl
from jax.experimental.pallas import tpu as pltpu
from jax.experimental.pallas import tpu_sc as plsc

B = 256
L = 200
NBLK = 13
TAIL_OFF = L - 16
K = 10
SIGMA = 1.0
NC = 2
NS = 16
NW = NC * NS
EX_PER_W = B // NW
LN2 = 0.6931471805599453
NEG_HUGE = -3.0e38

_DISC_LIST = [1.0 / math.log2(r + 2.0) for r in range(K)] + [0.0] * (16 - K)


def _blk(b):
    return pl.ds(16 * b if b < NBLK - 1 else TAIL_OFF, 16)


def _merge16(a, b):
    ak, av = a
    bk, bv = b
    bkr = lax.rev(bk, (0,))
    bvr = lax.rev(bv, (0,))
    take = ak >= bkr
    mk = jnp.where(take, ak, bkr)
    mv = jnp.where(take, av, bvr)
    return plsc.sort_key_val(mk, mv, descending=True)


def _top16(kv_blocks):
    runs = [plsc.sort_key_val(k, v, descending=True) for k, v in kv_blocks]
    while len(runs) > 1:
        nxt = [_merge16(runs[i], runs[i + 1]) for i in range(0, len(runs) - 1, 2)]
        if len(runs) % 2:
            nxt.append(runs[-1])
        runs = nxt
    return runs[0]


def _top16_vals_asc(key_blocks):
    runs = [jnp.sort(kb) for kb in key_blocks]
    while len(runs) > 1:
        nxt = []
        for i in range(0, len(runs) - 1, 2):
            mk = jnp.minimum(runs[i], lax.rev(runs[i + 1], (0,)))
            nxt.append(jnp.sort(mk))
        if len(runs) % 2:
            nxt.append(runs[-1])
        runs = nxt
    return runs[0]


def _sc_body(pred_hbm, targ_hbm, out_hbm, pred8_v, targ8_v, g_v, out_v):
    lane = lax.iota(jnp.int32, 16)
    zeros16 = (lane ^ lane).astype(jnp.float32)
    disc = zeros16
    for r in range(K):
        disc = jnp.where(lane == r, _DISC_LIST[r], disc)
    tail_ok = lane >= (16 * (NBLK - 1) - TAIL_OFF)
    wid = lax.axis_index("s") * NC + lax.axis_index("c")

    pltpu.sync_copy(pred_hbm.at[pl.ds(wid * EX_PER_W, EX_PER_W)], pred8_v)
    pltpu.sync_copy(targ_hbm.at[pl.ds(wid * EX_PER_W, EX_PER_W)], targ8_v)

    def ex_body(e, total):
        pj = [pred8_v[e, _blk(b)] for b in range(NBLK)]
        gj = [jnp.exp(targ8_v[e, _blk(b)] * LN2) - 1.0 for b in range(NBLK)]
        for b in range(NBLK):
            g_v[_blk(b)] = gj[b]
        pkey = pj[:-1] + [jnp.where(tail_ok, pj[-1], NEG_HUGE)]
        item = [lane + 16 * b for b in range(NBLK - 1)] + [lane + TAIL_OFF]
        pk, pv = _top16([(pkey[b], item[b]) for b in range(NBLK)])
        gkey = [-gj[b] for b in range(NBLK - 1)] + [
            jnp.where(tail_ok, -gj[-1], -NEG_HUGE)]
        gk_neg = _top16_vals_asc(gkey)
        ideal = jnp.broadcast_to(-jnp.sum(gk_neg * disc), (16,))
        recip = jnp.where(ideal == 0.0, zeros16, 1.0 / ideal)
        g_top = plsc.load_gather(g_v, [pv])

        acc = zeros16
        for t in range(K):
            sel = lane == t
            gt = jnp.broadcast_to(jnp.sum(jnp.where(sel, g_top, 0.0)), (16,))
            pt = jnp.broadcast_to(jnp.sum(jnp.where(sel, pk, 0.0)), (16,))
            dt = _DISC_LIST[t]
            sub_t = zeros16
            for b in range(NBLK):
                gd = gt - gj[b]
                pd = pt - pj[b]
                mis = gd * pd < 0.0
                if b == NBLK - 1:
                    mis = mis & tail_ok
                val = jnp.abs(gd) * (pt + pj[b])
                sub_t = sub_t + jnp.where(mis, val, 0.0)
            acc = acc + dt * sub_t
            gd = gt - g_top
            pd = pt - pk
            mis = (gd * pd < 0.0) & (lane < K)
            cval = (0.5 * jnp.abs(gd * (disc - dt)) - jnp.abs(gd) * dt) * (pt + pk)
            acc = acc + jnp.where(mis, cval, 0.0)

        return total + acc * (-SIGMA * recip)

    total = lax.fori_loop(0, EX_PER_W, ex_body, zeros16)
    out_v[...] = total
    pltpu.sync_copy(out_v, out_hbm.at[wid])


_sc_kernel = functools.partial(
    pl.kernel,
    out_type=jax.ShapeDtypeStruct((NW, 16), jnp.float32),
    mesh=plsc.VectorSubcoreMesh(
        core_axis_name="c", subcore_axis_name="s", num_cores=NC, num_subcores=NS
    ),
    compiler_params=pltpu.CompilerParams(
        needs_layout_passes=False, use_tc_tiling_on_sc=True
    ),
    scratch_types=[
        pltpu.VMEM((EX_PER_W, L), jnp.float32),
        pltpu.VMEM((EX_PER_W, L), jnp.float32),
        pltpu.VMEM((L,), jnp.float32),
        pltpu.VMEM((16,), jnp.float32),
    ],
)(_sc_body)


def kernel(predictions, targets):
    partials = _sc_kernel(predictions, targets)
    return jnp.sum(partials) / B


# speedup vs baseline: 155.6086x; 1.0206x over previous
import functools
import math

import jax
import jax.numpy as jnp
from jax import lax
from jax.experimental import pallas as pl
from jax.experimental.pallas import tpu as pltpu
from jax.experimental.pallas import tpu_sc as plsc

B = 256
L = 200
NBLK = 13
TAIL_OFF = L - 16
K = 10
SIGMA = 1.0
NC = 2
NS = 16
NW = NC * NS
EX_PER_W = B // NW
LN2 = 0.6931471805599453
NEG_HUGE = -3.0e38

_DISC_LIST = [1.0 / math.log2(r + 2.0) for r in range(K)] + [0.0] * (16 - K)


def _blk(b):
    return pl.ds(16 * b if b < NBLK - 1 else TAIL_OFF, 16)


def _merge16(a, b):
    ak, av = a
    bk, bv = b
    bkr = lax.rev(bk, (0,))
    bvr = lax.rev(bv, (0,))
    take = ak >= bkr
    mk = jnp.where(take, ak, bkr)
    mv = jnp.where(take, av, bvr)
    return plsc.sort_key_val(mk, mv, descending=True)


def _top16(kv_blocks):
    runs = [plsc.sort_key_val(k, v, descending=True) for k, v in kv_blocks]
    while len(runs) > 1:
        nxt = [_merge16(runs[i], runs[i + 1]) for i in range(0, len(runs) - 1, 2)]
        if len(runs) % 2:
            nxt.append(runs[-1])
        runs = nxt
    return runs[0]


def _top16_vals_asc(key_blocks):
    runs = [jnp.sort(kb) for kb in key_blocks]
    while len(runs) > 1:
        nxt = []
        for i in range(0, len(runs) - 1, 2):
            mk = jnp.minimum(runs[i], lax.rev(runs[i + 1], (0,)))
            nxt.append(jnp.sort(mk))
        if len(runs) % 2:
            nxt.append(runs[-1])
        runs = nxt
    return runs[0]


def _sc_body(pred_hbm, targ_hbm, out_hbm, pred8_v, targ8_v, g_v, out_v):
    lane = lax.iota(jnp.int32, 16)
    zeros16 = (lane ^ lane).astype(jnp.float32)
    disc = zeros16
    for r in range(K):
        disc = jnp.where(lane == r, _DISC_LIST[r], disc)
    tail_ok = lane >= (16 * (NBLK - 1) - TAIL_OFF)
    wid = lax.axis_index("s") * NC + lax.axis_index("c")

    pltpu.sync_copy(pred_hbm.at[pl.ds(wid * EX_PER_W, EX_PER_W)], pred8_v)
    pltpu.sync_copy(targ_hbm.at[pl.ds(wid * EX_PER_W, EX_PER_W)], targ8_v)

    def ex_body(e, total):
        pj = [pred8_v[e, _blk(b)] for b in range(NBLK)]
        gj = [jnp.exp(targ8_v[e, _blk(b)] * LN2) - 1.0 for b in range(NBLK)]
        for b in range(NBLK):
            g_v[_blk(b)] = gj[b]
        pkey = pj[:-1] + [jnp.where(tail_ok, pj[-1], NEG_HUGE)]
        item = [lane + 16 * b for b in range(NBLK - 1)] + [lane + TAIL_OFF]
        pk, pv = _top16([(pkey[b], item[b]) for b in range(NBLK)])
        gkey = [-gj[b] for b in range(NBLK - 1)] + [
            jnp.where(tail_ok, -gj[-1], -NEG_HUGE)]
        gk_neg = _top16_vals_asc(gkey)
        ideal = jnp.broadcast_to(-jnp.sum(gk_neg * disc), (16,))
        recip = jnp.where(ideal == 0.0, zeros16, 1.0 / ideal)
        g_top = plsc.load_gather(g_v, [pv])

        def t_body(t, acc):
            sel = lane == t
            gt = jnp.broadcast_to(jnp.sum(jnp.where(sel, g_top, 0.0)), (16,))
            pt = jnp.broadcast_to(jnp.sum(jnp.where(sel, pk, 0.0)), (16,))
            dt = jnp.broadcast_to(jnp.sum(jnp.where(sel, disc, 0.0)), (16,))
            sub_t = zeros16
            for b in range(NBLK):
                gd = gt - gj[b]
                pd = pt - pj[b]
                mis = gd * pd < 0.0
                if b == NBLK - 1:
                    mis = mis & tail_ok
                val = jnp.abs(gd) * (pt + pj[b])
                sub_t = sub_t + jnp.where(mis, val, 0.0)
            acc = acc + dt * sub_t
            gd = gt - g_top
            pd = pt - pk
            mis = (gd * pd < 0.0) & (lane < K)
            cval = (0.5 * jnp.abs(gd * (disc - dt)) - jnp.abs(gd) * dt) * (pt + pk)
            return acc + jnp.where(mis, cval, 0.0)

        acc = lax.fori_loop(0, K, t_body, zeros16)
        return total + acc * (-SIGMA * recip)

    total = lax.fori_loop(0, EX_PER_W, ex_body, zeros16)
    out_v[...] = total
    pltpu.sync_copy(out_v, out_hbm.at[wid])


_sc_kernel = functools.partial(
    pl.kernel,
    out_type=jax.ShapeDtypeStruct((NW, 16), jnp.float32),
    mesh=plsc.VectorSubcoreMesh(
        core_axis_name="c", subcore_axis_name="s", num_cores=NC, num_subcores=NS
    ),
    compiler_params=pltpu.CompilerParams(needs_layout_passes=False),
    scratch_types=[
        pltpu.VMEM((EX_PER_W, L), jnp.float32),
        pltpu.VMEM((EX_PER_W, L), jnp.float32),
        pltpu.VMEM((L,), jnp.float32),
        pltpu.VMEM((16,), jnp.float32),
    ],
)(_sc_body)


def kernel(predictions, targets):
    partials = _sc_kernel(predictions, targets)
    return jnp.sum(partials) / B


# speedup vs baseline: 159.2536x; 1.0234x over previous
import functools
import math

import jax
import jax.numpy as jnp
from jax import lax
from jax.experimental import pallas as pl
from jax.experimental.pallas import tpu as pltpu
from jax.experimental.pallas import tpu_sc as plsc

B = 256
L = 200
NBLK = 13
TAIL_OFF = L - 16
K = 10
SIGMA = 1.0
NC = 2
NS = 16
NW = NC * NS
EX_PER_W = B // NW
LN2 = 0.6931471805599453
NEG_HUGE = -3.0e38

_DISC_LIST = [1.0 / math.log2(r + 2.0) for r in range(K)] + [0.0] * (16 - K)


def _blk(b):
    return pl.ds(16 * b if b < NBLK - 1 else TAIL_OFF, 16)


def _merge16(a, b):
    ak, av = a
    bk, bv = b
    bkr = lax.rev(bk, (0,))
    bvr = lax.rev(bv, (0,))
    take = ak >= bkr
    mk = jnp.where(take, ak, bkr)
    mv = jnp.where(take, av, bvr)
    return plsc.sort_key_val(mk, mv, descending=True)


def _top16(kv_blocks):
    runs = [plsc.sort_key_val(k, v, descending=True) for k, v in kv_blocks]
    while len(runs) > 1:
        nxt = [_merge16(runs[i], runs[i + 1]) for i in range(0, len(runs) - 1, 2)]
        if len(runs) % 2:
            nxt.append(runs[-1])
        runs = nxt
    return runs[0]


def _top16_vals_asc(key_blocks):
    runs = [jnp.sort(kb) for kb in key_blocks]
    while len(runs) > 1:
        nxt = []
        for i in range(0, len(runs) - 1, 2):
            mk = jnp.minimum(runs[i], lax.rev(runs[i + 1], (0,)))
            nxt.append(jnp.sort(mk))
        if len(runs) % 2:
            nxt.append(runs[-1])
        runs = nxt
    return runs[0]


def _sc_body(pred_hbm, targ_hbm, out_hbm, pred8_v, targ8_v, g_v, out_v,
             sem_p, sem_t):
    lane = lax.iota(jnp.int32, 16)
    zeros16 = (lane ^ lane).astype(jnp.float32)
    disc = zeros16
    for r in range(K):
        disc = jnp.where(lane == r, _DISC_LIST[r], disc)
    tail_ok = lane >= (16 * (NBLK - 1) - TAIL_OFF)
    wid = lax.axis_index("s") * NC + lax.axis_index("c")

    cp_p = pltpu.async_copy(
        pred_hbm.at[pl.ds(wid * EX_PER_W, EX_PER_W)], pred8_v, sem_p)
    cp_t = pltpu.async_copy(
        targ_hbm.at[pl.ds(wid * EX_PER_W, EX_PER_W)], targ8_v, sem_t)
    cp_p.wait()
    cp_t.wait()

    def ex_body(e, total):
        pj = [pred8_v[e, _blk(b)] for b in range(NBLK)]
        gj = [jnp.exp(targ8_v[e, _blk(b)] * LN2) - 1.0 for b in range(NBLK)]
        for b in range(NBLK):
            g_v[_blk(b)] = gj[b]
        pkey = pj[:-1] + [jnp.where(tail_ok, pj[-1], NEG_HUGE)]
        item = [lane + 16 * b for b in range(NBLK - 1)] + [lane + TAIL_OFF]
        pk, pv = _top16([(pkey[b], item[b]) for b in range(NBLK)])
        gkey = [-gj[b] for b in range(NBLK - 1)] + [
            jnp.where(tail_ok, -gj[-1], -NEG_HUGE)]
        gk_neg = _top16_vals_asc(gkey)
        ideal = jnp.broadcast_to(-jnp.sum(gk_neg * disc), (16,))
        recip = jnp.where(ideal == 0.0, zeros16, 1.0 / ideal)
        g_top = plsc.load_gather(g_v, [pv])

        def t_body(t, acc):
            sel = lane == t
            gt = jnp.broadcast_to(jnp.sum(jnp.where(sel, g_top, 0.0)), (16,))
            pt = jnp.broadcast_to(jnp.sum(jnp.where(sel, pk, 0.0)), (16,))
            dt = jnp.broadcast_to(jnp.sum(jnp.where(sel, disc, 0.0)), (16,))
            sub_t = zeros16
            for b in range(NBLK):
                gd = gt - gj[b]
                pd = pt - pj[b]
                mis = gd * pd < 0.0
                if b == NBLK - 1:
                    mis = mis & tail_ok
                val = jnp.abs(gd) * (pt + pj[b])
                sub_t = sub_t + jnp.where(mis, val, 0.0)
            acc = acc + dt * sub_t
            gd = gt - g_top
            pd = pt - pk
            mis = (gd * pd < 0.0) & (lane < K)
            cval = (0.5 * jnp.abs(gd * (disc - dt)) - jnp.abs(gd) * dt) * (pt + pk)
            return acc + jnp.where(mis, cval, 0.0)

        acc = lax.fori_loop(0, K, t_body, zeros16)
        return total + acc * (-SIGMA * recip)

    total = lax.fori_loop(0, EX_PER_W, ex_body, zeros16)
    out_v[...] = total
    pltpu.sync_copy(out_v, out_hbm.at[wid])


_sc_kernel = functools.partial(
    pl.kernel,
    out_type=jax.ShapeDtypeStruct((NW, 16), jnp.float32),
    mesh=plsc.VectorSubcoreMesh(
        core_axis_name="c", subcore_axis_name="s", num_cores=NC, num_subcores=NS
    ),
    compiler_params=pltpu.CompilerParams(needs_layout_passes=False),
    scratch_types=[
        pltpu.VMEM((EX_PER_W, L), jnp.float32),
        pltpu.VMEM((EX_PER_W, L), jnp.float32),
        pltpu.VMEM((L,), jnp.float32),
        pltpu.VMEM((16,), jnp.float32),
        pltpu.SemaphoreType.DMA,
        pltpu.SemaphoreType.DMA,
    ],
)(_sc_body)


def kernel(predictions, targets):
    partials = _sc_kernel(predictions, targets)
    return jnp.sum(partials) / B
